# Initial kernel scaffold; baseline (speedup 1.0000x reference)
import jax
import jax.numpy as jnp
from jax.experimental import pallas as pl


def kernel(x, edge_index, W_gcn, b_gcn, W_ih, W_hh, b_ih, b_hh):
    raise NotImplementedError("write your pallas kernel here")


# speedup vs baseline: 4.1689x; 4.1689x over previous
import functools

import jax
import jax.numpy as jnp
from jax import lax
from jax.experimental import pallas as pl
from jax.experimental.pallas import tpu as pltpu

B = 64
T = 72
N = 5
F = 10
HG = 64
H = 320
E_PAD = 32
N_VALID = 21


def _fused_kernel(ed_ref, xt_ref, wg_ref, bg_ref, wih_ref, whh_ref,
                  bih_ref, bhh_ref, gout_ref, hout_ref, k_scr, h_scr):
    t = pl.program_id(0)

    @pl.when(t == 0)
    def _init():
        s_row = ed_ref[0:1, :]
        d_row = ed_ref[1:2, :]
        n_iota = lax.broadcasted_iota(jnp.int32, (8, E_PAD), 0)
        e_iota = lax.broadcasted_iota(jnp.int32, (8, E_PAD), 1)
        valid = (e_iota < N_VALID).astype(jnp.float32)
        oh_s = (jnp.broadcast_to(s_row, (8, E_PAD)) == n_iota)
        oh_d = (jnp.broadcast_to(d_row, (8, E_PAD)) == n_iota)
        oh_s = oh_s.astype(jnp.float32) * valid
        oh_d = oh_d.astype(jnp.float32) * valid
        deg = jnp.sum(oh_d, axis=1, keepdims=True)
        dis = jnp.where(deg > 0, lax.rsqrt(deg), 0.0)
        dis_s = jnp.sum(oh_s * dis, axis=0, keepdims=True)
        dis_d = jnp.sum(oh_d * dis, axis=0, keepdims=True)
        norm = dis_s * dis_d
        a8 = lax.dot_general(oh_d * norm, oh_s, (((1,), (1,)), ((), ())),
                             preferred_element_type=jnp.float32)

        r_i = lax.broadcasted_iota(jnp.int32, (N * F, 8), 0)
        c8_i = lax.broadcasted_iota(jnp.int32, (N * F, 8), 1)
        e_r = ((r_i // F) == c8_i).astype(jnp.float32)
        a_sel = lax.dot_general(e_r, a8, (((1,), (1,)), ((), ())),
                                preferred_element_type=jnp.float32)
        n8_i = lax.broadcasted_iota(jnp.int32, (8, H), 0)
        cH_i = lax.broadcasted_iota(jnp.int32, (8, H), 1)
        e_c = ((cH_i // HG) == n8_i).astype(jnp.float32)
        a_exp = jnp.dot(a_sel, e_c,
                        preferred_element_type=jnp.float32)
        rf_i = lax.broadcasted_iota(jnp.int32, (N * F, F), 0)
        cf_i = lax.broadcasted_iota(jnp.int32, (N * F, F), 1)
        f_r = ((rf_i % F) == cf_i).astype(jnp.float32)
        w_mid = jnp.dot(f_r, wg_ref[:],
                        preferred_element_type=jnp.float32)
        h_i = lax.broadcasted_iota(jnp.int32, (HG, H), 0)
        ch_i = lax.broadcasted_iota(jnp.int32, (HG, H), 1)
        f_c = ((ch_i % HG) == h_i).astype(jnp.float32)
        w_exp = jnp.dot(w_mid, f_c,
                        preferred_element_type=jnp.float32)
        k_scr[:] = a_exp * w_exp
        h_scr[:] = jnp.zeros((B, H), jnp.float32)

    xt = xt_ref[0]
    g = jnp.tanh(jnp.dot(xt, k_scr[:], preferred_element_type=jnp.float32)
                 + bg_ref[:])
    gout_ref[0] = g

    h = h_scr[:]
    gi = jnp.dot(g, wih_ref[:], preferred_element_type=jnp.float32) + bih_ref[:]
    gh = jnp.dot(h, whh_ref[:], preferred_element_type=jnp.float32) + bhh_ref[:]
    r = jax.nn.sigmoid(gi[:, 0:H] + gh[:, 0:H])
    z = jax.nn.sigmoid(gi[:, H:2 * H] + gh[:, H:2 * H])
    n = jnp.tanh(gi[:, 2 * H:3 * H] + r * gh[:, 2 * H:3 * H])
    h_new = (1.0 - z) * n + z * h
    h_scr[:] = h_new
    hout_ref[:] = h_new


@functools.partial(jax.jit, static_argnames=())
def kernel(x, edge_index, W_gcn, b_gcn, W_ih, W_hh, b_ih, b_hh):
    loops = jnp.arange(N, dtype=edge_index.dtype)
    s21 = jnp.concatenate([edge_index[0], loops])
    d21 = jnp.concatenate([edge_index[1], loops])
    ed = jnp.zeros((8, E_PAD), jnp.int32)
    ed = ed.at[0, :N_VALID].set(s21.astype(jnp.int32))
    ed = ed.at[1, :N_VALID].set(d21.astype(jnp.int32))

    xt = jnp.transpose(x, (1, 0, 2, 3)).reshape(T, B, N * F)
    bg_tiled = jnp.tile(b_gcn, N).reshape(1, H)
    wih_t = W_ih.T
    whh_t = W_hh.T
    bih = b_ih.reshape(1, 3 * H)
    bhh = b_hh.reshape(1, 3 * H)

    gout, h_last = pl.pallas_call(
        _fused_kernel,
        grid=(T,),
        in_specs=[
            pl.BlockSpec((8, E_PAD), lambda t: (0, 0)),
            pl.BlockSpec((1, B, N * F), lambda t: (t, 0, 0)),
            pl.BlockSpec((F, HG), lambda t: (0, 0)),
            pl.BlockSpec((1, H), lambda t: (0, 0)),
            pl.BlockSpec((H, 3 * H), lambda t: (0, 0)),
            pl.BlockSpec((H, 3 * H), lambda t: (0, 0)),
            pl.BlockSpec((1, 3 * H), lambda t: (0, 0)),
            pl.BlockSpec((1, 3 * H), lambda t: (0, 0)),
        ],
        out_specs=[
            pl.BlockSpec((1, B, H), lambda t: (t, 0, 0)),
            pl.BlockSpec((B, H), lambda t: (0, 0)),
        ],
        out_shape=[
            jax.ShapeDtypeStruct((T, B, H), jnp.float32),
            jax.ShapeDtypeStruct((B, H), jnp.float32),
        ],
        scratch_shapes=[
            pltpu.VMEM((N * F, H), jnp.float32),
            pltpu.VMEM((B, H), jnp.float32),
        ],
        compiler_params=pltpu.CompilerParams(
            dimension_semantics=("arbitrary",),
        ),
    )(ed, xt, W_gcn, bg_tiled, wih_t, whh_t, bih, bhh)

    gcn_features = jnp.transpose(gout, (1, 0, 2)).reshape(B, T, N, HG)
    return gcn_features, h_last
